# baseline (speedup 1.0000x reference)
import functools

import jax
import jax.numpy as jnp
from jax import lax
from jax.experimental import pallas as pl
from jax.experimental.pallas import tpu as pltpu
from jax.experimental.pallas import tpu_sc as plsc

N_NODES = 10000
D = 128
E = 320000

NC = 2
NS = 16
NW = NC * NS
CH_D = 128
NCH_D = 79
E_PAD_D = NW * NCH_D * CH_D
CH_S = 80
NCH_S = 125
E_PAD_S = NW * NCH_S * CH_S
N_PAD = 10112
RPT = N_PAD // NS

_MESH = plsc.VectorSubcoreMesh(core_axis_name="c", subcore_axis_name="s")


def _sc_degree(dst_t, ones_chunk, zeros_big):
    @functools.partial(
        pl.kernel,
        out_type=jax.ShapeDtypeStruct((NC * N_PAD, D), jnp.float32),
        mesh=_MESH,
        scratch_types=[
            pltpu.VMEM_SHARED((N_PAD, D), jnp.float32),
            pltpu.VMEM((NCH_D, CH_D), jnp.int32),
            pltpu.VMEM((CH_D, D), jnp.float32),
            pltpu.SemaphoreType.DMA,
        ],
    )
    def k(dst_hbm, ones_hbm, zeros_hbm, out_hbm, acc, idx_v, ones_v, sem):
        cid = lax.axis_index("c")
        sid = lax.axis_index("s")
        wid = sid * NC + cid
        pltpu.sync_copy(zeros_hbm.at[pl.ds(sid * RPT, RPT)],
                        acc.at[pl.ds(sid * RPT, RPT)])
        pltpu.sync_copy(dst_hbm.at[wid], idx_v)
        pltpu.sync_copy(ones_hbm, ones_v)
        plsc.subcore_barrier()

        @pl.loop(0, NCH_D)
        def _(j):
            pltpu.async_copy(ones_v, acc.at[idx_v.at[j]], sem, add=True)

        @pl.loop(0, NCH_D)
        def _(j):
            pltpu.make_async_copy(ones_v, acc.at[idx_v.at[j]], sem).wait()

        plsc.subcore_barrier()
        pltpu.sync_copy(acc.at[pl.ds(sid * RPT, RPT)],
                        out_hbm.at[pl.ds(cid * N_PAD + sid * RPT, RPT)])

    return k(dst_t, ones_chunk, zeros_big)


def _sc_segsum(h, idx_t, zeros_big):
    @functools.partial(
        pl.kernel,
        out_type=jax.ShapeDtypeStruct((NC * N_PAD, D), jnp.float32),
        mesh=_MESH,
        scratch_types=[
            pltpu.VMEM_SHARED((N_PAD, D), jnp.float32),
            pltpu.VMEM((8, 2, CH_S), jnp.int32),
            pltpu.VMEM((4, CH_S, D), jnp.float32),
        ] + [pltpu.SemaphoreType.DMA] * 14,
    )
    def k(h_hbm, idx_hbm, zeros_hbm, out_hbm, acc, rings, bufs, *sems):
        cid = lax.axis_index("c")
        sid = lax.axis_index("s")
        wid = sid * NC + cid
        gsems = sems[0:4]
        ssems = sems[4:6]
        isems = sems[6:14]
        pltpu.sync_copy(zeros_hbm.at[pl.ds(sid * RPT, RPT)],
                        acc.at[pl.ds(sid * RPT, RPT)])
        pltpu.sync_copy(idx_hbm.at[wid, 0], rings.at[0])
        plsc.subcore_barrier()

        for q in range(1, 7):
            pltpu.async_copy(idx_hbm.at[wid, q], rings.at[q], isems[q])
        pltpu.async_copy(h_hbm.at[rings.at[0].at[0]], bufs.at[0], gsems[0])
        for q in (1, 2):
            pltpu.make_async_copy(idx_hbm.at[wid, q], rings.at[q],
                                  isems[q]).wait()
            pltpu.async_copy(h_hbm.at[rings.at[q].at[0]], bufs.at[q],
                             gsems[q])

        @pl.loop(0, NCH_S, step=8)
        def _(j):
            for kk in range(8):
                @pl.when(j + kk < NCH_S)
                def _(kk=kk):
                    c = j + kk
                    rg = rings.at[kk]
                    bf = bufs.at[kk % 4]
                    pltpu.make_async_copy(h_hbm.at[rg.at[0]], bf,
                                          gsems[kk % 4]).wait()
                    pltpu.async_copy(bf, acc.at[rg.at[1]], ssems[kk % 2],
                                     add=True)

                    @pl.when(c + 3 < NCH_S)
                    def _():
                        pltpu.make_async_copy(
                            idx_hbm.at[wid, c + 3],
                            rings.at[(kk + 3) % 8],
                            isems[(kk + 3) % 8]).wait()

                        @pl.when(c >= 1)
                        def _():
                            pltpu.make_async_copy(
                                bufs.at[(kk + 3) % 4], acc.at[rg.at[1]],
                                ssems[(kk + 1) % 2]).wait()
                        pltpu.async_copy(h_hbm.at[rings.at[(kk + 3) % 8].at[0]],
                                        bufs.at[(kk + 3) % 4],
                                        gsems[(kk + 3) % 4])

                    @pl.when(c + 7 < NCH_S)
                    def _():
                        pltpu.async_copy(idx_hbm.at[wid, c + 7],
                                         rings.at[(kk + 7) % 8],
                                         isems[(kk + 7) % 8])

        for q in (4, 3, 2, 1):
            pltpu.make_async_copy(bufs.at[(NCH_S - q) % 4],
                                  acc.at[rings.at[0].at[1]],
                                  ssems[(NCH_S - q) % 2]).wait()
        plsc.subcore_barrier()
        pltpu.sync_copy(acc.at[pl.ds(sid * RPT, RPT)],
                        out_hbm.at[pl.ds(cid * N_PAD + sid * RPT, RPT)])

    return k(h, idx_t, zeros_big)


def _tc_matmul_scale(x, W, deg_p):
    def body(x_ref, w_ref, deg_ref, h1p_ref, dis_ref):
        h = lax.dot_general(
            x_ref[...], w_ref[...], (((1,), (0,)), ((), ())),
            precision=lax.Precision.HIGHEST,
            preferred_element_type=jnp.float32)
        deg = (deg_ref[0:N_NODES, 0:1]
               + deg_ref[N_PAD:N_PAD + N_NODES, 0:1] + 1.0)
        dis = lax.rsqrt(deg)
        dis_ref[...] = dis
        h1p_ref[...] = h * dis

    return pl.pallas_call(
        body,
        out_shape=(jax.ShapeDtypeStruct((N_NODES, D), jnp.float32),
                   jax.ShapeDtypeStruct((N_NODES, 1), jnp.float32)),
    )(x, W, deg_p)


def _tc_mid(s1, h1p, dis, b1, W2):
    def body(s_ref, h1p_ref, dis_ref, b1_ref, w2_ref, o_ref):
        s = (s_ref[0:N_NODES, :] + s_ref[N_PAD:N_PAD + N_NODES, :]
             + h1p_ref[...])
        z = dis_ref[...] * s + b1_ref[...][None, :]
        x2 = jnp.where(z >= 0, z, 0.01 * z)
        h2 = lax.dot_general(
            x2, w2_ref[...], (((1,), (0,)), ((), ())),
            precision=lax.Precision.HIGHEST,
            preferred_element_type=jnp.float32)
        o_ref[...] = h2 * dis_ref[...]

    return pl.pallas_call(
        body,
        out_shape=jax.ShapeDtypeStruct((N_NODES, D), jnp.float32),
    )(s1, h1p, dis, b1, W2)


def _tc_final(s2, h2p, dis, b2):
    def body(s_ref, h2p_ref, dis_ref, b2_ref, o_ref):
        s = (s_ref[0:N_NODES, :] + s_ref[N_PAD:N_PAD + N_NODES, :]
             + h2p_ref[...])
        o_ref[...] = dis_ref[...] * s + b2_ref[...][None, :]

    return pl.pallas_call(
        body,
        out_shape=jax.ShapeDtypeStruct((N_NODES, D), jnp.float32),
    )(s2, h2p, dis, b2)


def kernel(x, edge_index, W1, b1, W2, b2):
    src = edge_index[0].astype(jnp.int32)
    dst = edge_index[1].astype(jnp.int32)

    def pad_edges(v, e_pad, base):
        npad = e_pad - E
        pad_ar = jnp.arange(npad, dtype=jnp.int32)
        fill = pad_ar % base[0] + base[1]
        return jnp.concatenate([v, fill])

    dst_t = pad_edges(dst, E_PAD_D,
                      (N_PAD - N_NODES, N_NODES)).reshape(NW, NCH_D, CH_D)
    src_s = pad_edges(src, E_PAD_S, (N_NODES, 0)).reshape(NW, NCH_S, CH_S)
    dst_s = pad_edges(dst, E_PAD_S,
                      (N_PAD - N_NODES, N_NODES)).reshape(NW, NCH_S, CH_S)
    idx_t = jnp.stack([src_s, dst_s], axis=2)

    ones_chunk = jnp.ones((CH_D, D), jnp.float32)
    zeros_big = jnp.zeros((N_PAD, D), jnp.float32)

    deg_p = _sc_degree(dst_t, ones_chunk, zeros_big)
    h1p, dis = _tc_matmul_scale(x, W1, deg_p)
    s1 = _sc_segsum(h1p, idx_t, zeros_big)
    h2p = _tc_mid(s1, h1p, dis, b1, W2)
    s2 = _sc_segsum(h2p, idx_t, zeros_big)
    return _tc_final(s2, h2p, dis, b2)


# speedup vs baseline: 1.0138x; 1.0138x over previous
import functools

import jax
import jax.numpy as jnp
from jax import lax
from jax.experimental import pallas as pl
from jax.experimental.pallas import tpu as pltpu
from jax.experimental.pallas import tpu_sc as plsc

N_NODES = 10000
D = 128
E = 320000

NC = 2
NS = 16
NW = NC * NS
CH_D = 128
NCH_D = 79
E_PAD_D = NW * NCH_D * CH_D
CH_S = 112
NCH_S = 90
E_PAD_S = NW * NCH_S * CH_S
N_PAD = 10112
RPT = N_PAD // NS

_MESH = plsc.VectorSubcoreMesh(core_axis_name="c", subcore_axis_name="s")


def _sc_degree(dst_t, ones_chunk, zeros_big):
    @functools.partial(
        pl.kernel,
        out_type=jax.ShapeDtypeStruct((NC * N_PAD, D), jnp.float32),
        mesh=_MESH,
        scratch_types=[
            pltpu.VMEM_SHARED((N_PAD, D), jnp.float32),
            pltpu.VMEM((NCH_D, CH_D), jnp.int32),
            pltpu.VMEM((CH_D, D), jnp.float32),
            pltpu.SemaphoreType.DMA,
            pltpu.SemaphoreType.DMA,
        ],
    )
    def k(dst_hbm, ones_hbm, zeros_hbm, out_hbm, acc, idx_v, ones_v,
          sem, lsem):
        cid = lax.axis_index("c")
        sid = lax.axis_index("s")
        wid = sid * NC + cid
        pltpu.async_copy(dst_hbm.at[wid], idx_v, lsem)
        pltpu.async_copy(ones_hbm, ones_v, lsem)
        pltpu.sync_copy(zeros_hbm.at[pl.ds(sid * RPT, RPT)],
                        acc.at[pl.ds(sid * RPT, RPT)])
        pltpu.make_async_copy(dst_hbm.at[wid], idx_v, lsem).wait()
        pltpu.make_async_copy(ones_hbm, ones_v, lsem).wait()
        plsc.subcore_barrier()

        @pl.loop(0, NCH_D)
        def _(j):
            pltpu.async_copy(ones_v, acc.at[idx_v.at[j]], sem, add=True)

        @pl.loop(0, NCH_D)
        def _(j):
            pltpu.make_async_copy(ones_v, acc.at[idx_v.at[j]], sem).wait()

        plsc.subcore_barrier()
        pltpu.sync_copy(acc.at[pl.ds(sid * RPT, RPT)],
                        out_hbm.at[pl.ds(cid * N_PAD + sid * RPT, RPT)])

    return k(dst_t, ones_chunk, zeros_big)


def _sc_segsum(h, idx_t, zeros_big):
    @functools.partial(
        pl.kernel,
        out_type=jax.ShapeDtypeStruct((NC * N_PAD, D), jnp.float32),
        mesh=_MESH,
        scratch_types=[
            pltpu.VMEM_SHARED((N_PAD, D), jnp.float32),
            pltpu.VMEM((6, 2, CH_S), jnp.int32),
            pltpu.VMEM((3, CH_S, D), jnp.float32),
        ] + [pltpu.SemaphoreType.DMA] * 11,
    )
    def k(h_hbm, idx_hbm, zeros_hbm, out_hbm, acc, rings, bufs, *sems):
        cid = lax.axis_index("c")
        sid = lax.axis_index("s")
        wid = sid * NC + cid
        gsems = sems[0:3]
        ssems = sems[3:5]
        isems = sems[5:11]
        for q in range(1, 5):
            pltpu.async_copy(idx_hbm.at[wid, q], rings.at[q], isems[q])
        pltpu.sync_copy(idx_hbm.at[wid, 0], rings.at[0])
        pltpu.async_copy(h_hbm.at[rings.at[0].at[0]], bufs.at[0], gsems[0])
        pltpu.make_async_copy(idx_hbm.at[wid, 1], rings.at[1],
                              isems[1]).wait()
        pltpu.async_copy(h_hbm.at[rings.at[1].at[0]], bufs.at[1], gsems[1])
        pltpu.sync_copy(zeros_hbm.at[pl.ds(sid * RPT, RPT)],
                        acc.at[pl.ds(sid * RPT, RPT)])
        plsc.subcore_barrier()

        @pl.loop(0, NCH_S, step=6)
        def _(j):
            for kk in range(6):
                @pl.when(j + kk < NCH_S)
                def _(kk=kk):
                    c = j + kk
                    rg = rings.at[kk]
                    bf = bufs.at[kk % 3]
                    pltpu.make_async_copy(h_hbm.at[rg.at[0]], bf,
                                          gsems[kk % 3]).wait()
                    pltpu.async_copy(bf, acc.at[rg.at[1]], ssems[kk % 2],
                                     add=True)

                    @pl.when(c + 2 < NCH_S)
                    def _():
                        pltpu.make_async_copy(
                            idx_hbm.at[wid, c + 2],
                            rings.at[(kk + 2) % 6],
                            isems[(kk + 2) % 6]).wait()

                        @pl.when(c >= 1)
                        def _():
                            pltpu.make_async_copy(
                                bufs.at[(kk + 2) % 3], acc.at[rg.at[1]],
                                ssems[(kk + 1) % 2]).wait()
                        pltpu.async_copy(h_hbm.at[rings.at[(kk + 2) % 6].at[0]],
                                        bufs.at[(kk + 2) % 3],
                                        gsems[(kk + 2) % 3])

                    @pl.when(c + 5 < NCH_S)
                    def _():
                        pltpu.async_copy(idx_hbm.at[wid, c + 5],
                                         rings.at[(kk + 5) % 6],
                                         isems[(kk + 5) % 6])

        for q in (3, 2, 1):
            pltpu.make_async_copy(bufs.at[(NCH_S - q) % 3],
                                  acc.at[rings.at[0].at[1]],
                                  ssems[(NCH_S - q) % 2]).wait()
        plsc.subcore_barrier()
        pltpu.sync_copy(acc.at[pl.ds(sid * RPT, RPT)],
                        out_hbm.at[pl.ds(cid * N_PAD + sid * RPT, RPT)])

    return k(h, idx_t, zeros_big)


def _tc_matmul_scale(x, W, deg_p):
    def body(x_ref, w_ref, deg_ref, h1p_ref, dis_ref):
        h = lax.dot_general(
            x_ref[...], w_ref[...], (((1,), (0,)), ((), ())),
            precision=lax.Precision.HIGHEST,
            preferred_element_type=jnp.float32)
        deg = (deg_ref[0:N_NODES, 0:1]
               + deg_ref[N_PAD:N_PAD + N_NODES, 0:1] + 1.0)
        dis = lax.rsqrt(deg)
        dis_ref[...] = dis
        h1p_ref[...] = h * dis

    return pl.pallas_call(
        body,
        out_shape=(jax.ShapeDtypeStruct((N_NODES, D), jnp.float32),
                   jax.ShapeDtypeStruct((N_NODES, 1), jnp.float32)),
    )(x, W, deg_p)


def _tc_mid(s1, h1p, dis, b1, W2):
    def body(s_ref, h1p_ref, dis_ref, b1_ref, w2_ref, o_ref):
        s = (s_ref[0:N_NODES, :] + s_ref[N_PAD:N_PAD + N_NODES, :]
             + h1p_ref[...])
        z = dis_ref[...] * s + b1_ref[...][None, :]
        x2 = jnp.where(z >= 0, z, 0.01 * z)
        h2 = lax.dot_general(
            x2, w2_ref[...], (((1,), (0,)), ((), ())),
            precision=lax.Precision.HIGHEST,
            preferred_element_type=jnp.float32)
        o_ref[...] = h2 * dis_ref[...]

    return pl.pallas_call(
        body,
        out_shape=jax.ShapeDtypeStruct((N_NODES, D), jnp.float32),
    )(s1, h1p, dis, b1, W2)


def _tc_final(s2, h2p, dis, b2):
    def body(s_ref, h2p_ref, dis_ref, b2_ref, o_ref):
        s = (s_ref[0:N_NODES, :] + s_ref[N_PAD:N_PAD + N_NODES, :]
             + h2p_ref[...])
        o_ref[...] = dis_ref[...] * s + b2_ref[...][None, :]

    return pl.pallas_call(
        body,
        out_shape=jax.ShapeDtypeStruct((N_NODES, D), jnp.float32),
    )(s2, h2p, dis, b2)


def kernel(x, edge_index, W1, b1, W2, b2):
    src = edge_index[0].astype(jnp.int32)
    dst = edge_index[1].astype(jnp.int32)

    def pad_edges(v, e_pad, base):
        npad = e_pad - E
        pad_ar = jnp.arange(npad, dtype=jnp.int32)
        fill = pad_ar % base[0] + base[1]
        return jnp.concatenate([v, fill])

    dst_t = pad_edges(dst, E_PAD_D,
                      (N_PAD - N_NODES, N_NODES)).reshape(NW, NCH_D, CH_D)
    src_s = pad_edges(src, E_PAD_S, (N_NODES, 0)).reshape(NW, NCH_S, CH_S)
    dst_s = pad_edges(dst, E_PAD_S,
                      (N_PAD - N_NODES, N_NODES)).reshape(NW, NCH_S, CH_S)
    idx_t = jnp.stack([src_s, dst_s], axis=2)

    ones_chunk = jnp.ones((CH_D, D), jnp.float32)
    zeros_big = jnp.zeros((N_PAD, D), jnp.float32)

    deg_p = _sc_degree(dst_t, ones_chunk, zeros_big)
    h1p, dis = _tc_matmul_scale(x, W1, deg_p)
    s1 = _sc_segsum(h1p, idx_t, zeros_big)
    h2p = _tc_mid(s1, h1p, dis, b1, W2)
    s2 = _sc_segsum(h2p, idx_t, zeros_big)
    return _tc_final(s2, h2p, dis, b2)


# speedup vs baseline: 1.1779x; 1.1619x over previous
import dataclasses
import functools

import jax
import jax.numpy as jnp
from jax import lax
from jax.experimental import pallas as pl
from jax.experimental.pallas import tpu as pltpu
from jax.experimental.pallas import tpu_sc as plsc

N_NODES = 10000
D = 128
E = 320000

NC = 2
NS = 16
NW = NC * NS
CH_D = 128
NCH_D = 79
E_PAD_D = NW * NCH_D * CH_D
CH_S = 112
NCH_S = 90
E_PAD_S = NW * NCH_S * CH_S
N_PAD = 10112
RPT = N_PAD // NS

_MESH = plsc.VectorSubcoreMesh(core_axis_name="c", subcore_axis_name="s")

_CP_NO_LAYOUT = pltpu.CompilerParams()
if "needs_layout_passes" in pltpu.CompilerParams.__dataclass_fields__:
    _CP_NO_LAYOUT = dataclasses.replace(_CP_NO_LAYOUT,
                                        needs_layout_passes=False)


def _sc_degree(dst_t, zeros_loc):
    @functools.partial(
        pl.kernel,
        out_type=jax.ShapeDtypeStruct((NW, NCH_D, CH_D), jnp.float32),
        mesh=_MESH,
        scratch_types=[
            pltpu.VMEM((NCH_D, CH_D), jnp.int32),
            pltpu.VMEM((NCH_D, CH_D), jnp.float32),
            pltpu.SemaphoreType.DMA,
        ],
        compiler_params=_CP_NO_LAYOUT,
    )
    def k(dst_hbm, zeros_hbm, out_hbm, idx_v, acc_v, lsem):
        cid = lax.axis_index("c")
        sid = lax.axis_index("s")
        wid = sid * NC + cid
        pltpu.async_copy(dst_hbm.at[wid], idx_v, lsem)
        pltpu.sync_copy(zeros_hbm, acc_v)
        pltpu.make_async_copy(dst_hbm.at[wid], idx_v, lsem).wait()
        ones = jnp.ones((16,), jnp.float32)

        @pl.loop(0, NCH_D)
        def _(r):
            @pl.loop(0, CH_D, step=16)
            def _(i):
                v = idx_v[r, pl.ds(i, 16)]
                hi = lax.shift_right_logical(v, 7)
                lo = lax.bitwise_and(v, 127)
                plsc.addupdate_scatter(acc_v, [hi, lo], ones)

        pltpu.sync_copy(acc_v, out_hbm.at[wid])

    return k(dst_t, zeros_loc)


def _sc_segsum(h, idx_t, zeros_big):
    @functools.partial(
        pl.kernel,
        out_type=jax.ShapeDtypeStruct((NC * N_PAD, D), jnp.float32),
        mesh=_MESH,
        scratch_types=[
            pltpu.VMEM_SHARED((N_PAD, D), jnp.float32),
            pltpu.VMEM((6, 2, CH_S), jnp.int32),
            pltpu.VMEM((3, CH_S, D), jnp.float32),
        ] + [pltpu.SemaphoreType.DMA] * 11,
    )
    def k(h_hbm, idx_hbm, zeros_hbm, out_hbm, acc, rings, bufs, *sems):
        cid = lax.axis_index("c")
        sid = lax.axis_index("s")
        wid = sid * NC + cid
        gsems = sems[0:3]
        ssems = sems[3:5]
        isems = sems[5:11]
        for q in range(1, 5):
            pltpu.async_copy(idx_hbm.at[wid, q], rings.at[q], isems[q])
        pltpu.sync_copy(idx_hbm.at[wid, 0], rings.at[0])
        pltpu.async_copy(h_hbm.at[rings.at[0].at[0]], bufs.at[0], gsems[0])
        pltpu.make_async_copy(idx_hbm.at[wid, 1], rings.at[1],
                              isems[1]).wait()
        pltpu.async_copy(h_hbm.at[rings.at[1].at[0]], bufs.at[1], gsems[1])
        pltpu.sync_copy(zeros_hbm.at[pl.ds(sid * RPT, RPT)],
                        acc.at[pl.ds(sid * RPT, RPT)])
        plsc.subcore_barrier()

        @pl.loop(0, NCH_S, step=6)
        def _(j):
            for kk in range(6):
                @pl.when(j + kk < NCH_S)
                def _(kk=kk):
                    c = j + kk
                    rg = rings.at[kk]
                    bf = bufs.at[kk % 3]
                    pltpu.make_async_copy(h_hbm.at[rg.at[0]], bf,
                                          gsems[kk % 3]).wait()
                    pltpu.async_copy(bf, acc.at[rg.at[1]], ssems[kk % 2],
                                     add=True)

                    @pl.when(c + 2 < NCH_S)
                    def _():
                        pltpu.make_async_copy(
                            idx_hbm.at[wid, c + 2],
                            rings.at[(kk + 2) % 6],
                            isems[(kk + 2) % 6]).wait()

                        @pl.when(c >= 1)
                        def _():
                            pltpu.make_async_copy(
                                bufs.at[(kk + 2) % 3], acc.at[rg.at[1]],
                                ssems[(kk + 1) % 2]).wait()
                        pltpu.async_copy(h_hbm.at[rings.at[(kk + 2) % 6].at[0]],
                                        bufs.at[(kk + 2) % 3],
                                        gsems[(kk + 2) % 3])

                    @pl.when(c + 5 < NCH_S)
                    def _():
                        pltpu.async_copy(idx_hbm.at[wid, c + 5],
                                         rings.at[(kk + 5) % 6],
                                         isems[(kk + 5) % 6])

        for q in (3, 2, 1):
            pltpu.make_async_copy(bufs.at[(NCH_S - q) % 3],
                                  acc.at[rings.at[0].at[1]],
                                  ssems[(NCH_S - q) % 2]).wait()
        plsc.subcore_barrier()
        pltpu.sync_copy(acc.at[pl.ds(sid * RPT, RPT)],
                        out_hbm.at[pl.ds(cid * N_PAD + sid * RPT, RPT)])

    return k(h, idx_t, zeros_big)


def _tc_dis(deg_p):
    def body(deg_ref, dis_ref):
        dis_ref[...] = lax.rsqrt(jnp.sum(deg_ref[...], axis=0) + 1.0)

    return pl.pallas_call(
        body,
        out_shape=jax.ShapeDtypeStruct((NCH_D * CH_D,), jnp.float32),
    )(deg_p)


def _tc_matmul_scale(x, W, dis):
    def body(x_ref, w_ref, dis_ref, h1p_ref):
        h = lax.dot_general(
            x_ref[...], w_ref[...], (((1,), (0,)), ((), ())),
            precision=lax.Precision.HIGHEST,
            preferred_element_type=jnp.float32)
        h1p_ref[...] = h * dis_ref[...]

    return pl.pallas_call(
        body,
        out_shape=jax.ShapeDtypeStruct((N_NODES, D), jnp.float32),
    )(x, W, dis)


def _tc_mid(s1, h1p, dis, b1, W2):
    def body(s_ref, h1p_ref, dis_ref, b1_ref, w2_ref, o_ref):
        s = (s_ref[0:N_NODES, :] + s_ref[N_PAD:N_PAD + N_NODES, :]
             + h1p_ref[...])
        z = dis_ref[...] * s + b1_ref[...][None, :]
        x2 = jnp.where(z >= 0, z, 0.01 * z)
        h2 = lax.dot_general(
            x2, w2_ref[...], (((1,), (0,)), ((), ())),
            precision=lax.Precision.HIGHEST,
            preferred_element_type=jnp.float32)
        o_ref[...] = h2 * dis_ref[...]

    return pl.pallas_call(
        body,
        out_shape=jax.ShapeDtypeStruct((N_NODES, D), jnp.float32),
    )(s1, h1p, dis, b1, W2)


def _tc_final(s2, h2p, dis, b2):
    def body(s_ref, h2p_ref, dis_ref, b2_ref, o_ref):
        s = (s_ref[0:N_NODES, :] + s_ref[N_PAD:N_PAD + N_NODES, :]
             + h2p_ref[...])
        o_ref[...] = dis_ref[...] * s + b2_ref[...][None, :]

    return pl.pallas_call(
        body,
        out_shape=jax.ShapeDtypeStruct((N_NODES, D), jnp.float32),
    )(s2, h2p, dis, b2)


def kernel(x, edge_index, W1, b1, W2, b2):
    src = edge_index[0].astype(jnp.int32)
    dst = edge_index[1].astype(jnp.int32)

    def pad_edges(v, e_pad, base):
        npad = e_pad - E
        pad_ar = jnp.arange(npad, dtype=jnp.int32)
        fill = pad_ar % base[0] + base[1]
        return jnp.concatenate([v, fill])

    dst_t = pad_edges(dst, E_PAD_D,
                      (N_PAD - N_NODES, N_NODES)).reshape(NW, NCH_D, CH_D)
    src_s = pad_edges(src, E_PAD_S, (N_NODES, 0)).reshape(NW, NCH_S, CH_S)
    dst_s = pad_edges(dst, E_PAD_S,
                      (N_PAD - N_NODES, N_NODES)).reshape(NW, NCH_S, CH_S)
    idx_t = jnp.stack([src_s, dst_s], axis=2)

    zeros_big = jnp.zeros((N_PAD, D), jnp.float32)
    zeros_loc = jnp.zeros((NCH_D, CH_D), jnp.float32)

    deg_p = _sc_degree(dst_t, zeros_loc)
    dis_flat = _tc_dis(deg_p.reshape(NW, NCH_D * CH_D))
    dis = dis_flat.reshape(-1, 1)[:N_NODES]
    h1p = _tc_matmul_scale(x, W1, dis)
    s1 = _sc_segsum(h1p, idx_t, zeros_big)
    h2p = _tc_mid(s1, h1p, dis, b1, W2)
    s2 = _sc_segsum(h2p, idx_t, zeros_big)
    return _tc_final(s2, h2p, dis, b2)


# speedup vs baseline: 1.2060x; 1.0238x over previous
import dataclasses
import functools

import jax
import jax.numpy as jnp
from jax import lax
from jax.experimental import pallas as pl
from jax.experimental.pallas import tpu as pltpu
from jax.experimental.pallas import tpu_sc as plsc

N_NODES = 10000
D = 128
E = 320000

NC = 2
NS = 16
NW = NC * NS
CH_D = 128
NCH_D = 79
E_PAD_D = NW * NCH_D * CH_D
CH_S = 112
NCH_S = 90
E_PAD_S = NW * NCH_S * CH_S
N_PAD = 10112
RPT = N_PAD // NS

_MESH = plsc.VectorSubcoreMesh(core_axis_name="c", subcore_axis_name="s")

_CP_NO_LAYOUT = pltpu.CompilerParams()
if "needs_layout_passes" in pltpu.CompilerParams.__dataclass_fields__:
    _CP_NO_LAYOUT = dataclasses.replace(_CP_NO_LAYOUT,
                                        needs_layout_passes=False)


def _sc_degree(dst_t, zeros_loc):
    @functools.partial(
        pl.kernel,
        out_type=jax.ShapeDtypeStruct((NW, NCH_D, CH_D), jnp.float32),
        mesh=_MESH,
        scratch_types=[
            pltpu.VMEM((NCH_D, CH_D), jnp.int32),
            pltpu.VMEM((NCH_D, CH_D), jnp.float32),
            pltpu.SemaphoreType.DMA,
        ],
        compiler_params=_CP_NO_LAYOUT,
    )
    def k(dst_hbm, zeros_hbm, out_hbm, idx_v, acc_v, lsem):
        cid = lax.axis_index("c")
        sid = lax.axis_index("s")
        wid = sid * NC + cid
        pltpu.async_copy(dst_hbm.at[wid], idx_v, lsem)
        pltpu.sync_copy(zeros_hbm, acc_v)
        pltpu.make_async_copy(dst_hbm.at[wid], idx_v, lsem).wait()
        ones = jnp.ones((16,), jnp.float32)

        @pl.loop(0, NCH_D)
        def _(r):
            @pl.loop(0, CH_D, step=16)
            def _(i):
                v = idx_v[r, pl.ds(i, 16)]
                hi = lax.shift_right_logical(v, 7)
                lo = lax.bitwise_and(v, 127)
                plsc.addupdate_scatter(acc_v, [hi, lo], ones)

        pltpu.sync_copy(acc_v, out_hbm.at[wid])

    return k(dst_t, zeros_loc)


def _sc_segsum(h, idx_t, zeros_big):
    @functools.partial(
        pl.kernel,
        out_type=jax.ShapeDtypeStruct((NC * N_PAD, D), jnp.float32),
        mesh=_MESH,
        scratch_types=[
            pltpu.VMEM_SHARED((N_PAD, D), jnp.float32),
            pltpu.VMEM((6, 2, CH_S), jnp.int32),
            pltpu.VMEM((3, CH_S, D), jnp.float32),
        ] + [pltpu.SemaphoreType.DMA] * 11,
    )
    def k(h_hbm, idx_hbm, zeros_hbm, out_hbm, acc, rings, bufs, *sems):
        cid = lax.axis_index("c")
        sid = lax.axis_index("s")
        wid = sid * NC + cid
        gsems = sems[0:3]
        ssems = sems[3:5]
        isems = sems[5:11]
        for q in range(1, 5):
            pltpu.async_copy(idx_hbm.at[wid, q], rings.at[q], isems[q])
        pltpu.sync_copy(idx_hbm.at[wid, 0], rings.at[0])
        pltpu.async_copy(h_hbm.at[rings.at[0].at[0]], bufs.at[0], gsems[0])
        pltpu.make_async_copy(idx_hbm.at[wid, 1], rings.at[1],
                              isems[1]).wait()
        pltpu.async_copy(h_hbm.at[rings.at[1].at[0]], bufs.at[1], gsems[1])
        pltpu.sync_copy(zeros_hbm.at[pl.ds(sid * RPT, RPT)],
                        acc.at[pl.ds(sid * RPT, RPT)])
        plsc.subcore_barrier()

        @pl.loop(0, NCH_S, step=6)
        def _(j):
            for kk in range(6):
                @pl.when(j + kk < NCH_S)
                def _(kk=kk):
                    c = j + kk
                    rg = rings.at[kk]
                    bf = bufs.at[kk % 3]
                    pltpu.make_async_copy(h_hbm.at[rg.at[0]], bf,
                                          gsems[kk % 3]).wait()
                    pltpu.async_copy(bf, acc.at[rg.at[1]], ssems[kk % 2],
                                     add=True)

                    @pl.when(c + 2 < NCH_S)
                    def _():
                        pltpu.make_async_copy(
                            idx_hbm.at[wid, c + 2],
                            rings.at[(kk + 2) % 6],
                            isems[(kk + 2) % 6]).wait()

                        @pl.when(c >= 1)
                        def _():
                            pltpu.make_async_copy(
                                bufs.at[(kk + 2) % 3], acc.at[rg.at[1]],
                                ssems[(kk + 1) % 2]).wait()
                        pltpu.async_copy(h_hbm.at[rings.at[(kk + 2) % 6].at[0]],
                                        bufs.at[(kk + 2) % 3],
                                        gsems[(kk + 2) % 3])

                    @pl.when(c + 5 < NCH_S)
                    def _():
                        pltpu.async_copy(idx_hbm.at[wid, c + 5],
                                         rings.at[(kk + 5) % 6],
                                         isems[(kk + 5) % 6])

        for q in (3, 2, 1):
            pltpu.make_async_copy(bufs.at[(NCH_S - q) % 3],
                                  acc.at[rings.at[0].at[1]],
                                  ssems[(NCH_S - q) % 2]).wait()
        plsc.subcore_barrier()
        pltpu.sync_copy(acc.at[pl.ds(sid * RPT, RPT)],
                        out_hbm.at[pl.ds(cid * N_PAD + sid * RPT, RPT)])

    return k(h, idx_t, zeros_big)


def _tc_matmul_scale(x, W, deg_p):
    def body(x_ref, w_ref, deg_ref, h1p_ref, dis_ref):
        h = lax.dot_general(
            x_ref[...], w_ref[...], (((1,), (0,)), ((), ())),
            precision=lax.Precision.HIGHEST,
            preferred_element_type=jnp.float32)
        disf = lax.rsqrt(jnp.sum(deg_ref[...], axis=0) + 1.0)
        dis = jnp.reshape(disf, (NCH_D * CH_D, 1))[0:N_NODES]
        dis_ref[...] = dis
        h1p_ref[...] = h * dis

    return pl.pallas_call(
        body,
        out_shape=(jax.ShapeDtypeStruct((N_NODES, D), jnp.float32),
                   jax.ShapeDtypeStruct((N_NODES, 1), jnp.float32)),
    )(x, W, deg_p)


def _tc_mid(s1, h1p, dis, b1, W2):
    def body(s_ref, h1p_ref, dis_ref, b1_ref, w2_ref, o_ref):
        s = (s_ref[0:N_NODES, :] + s_ref[N_PAD:N_PAD + N_NODES, :]
             + h1p_ref[...])
        z = dis_ref[...] * s + b1_ref[...][None, :]
        x2 = jnp.where(z >= 0, z, 0.01 * z)
        h2 = lax.dot_general(
            x2, w2_ref[...], (((1,), (0,)), ((), ())),
            precision=lax.Precision.HIGHEST,
            preferred_element_type=jnp.float32)
        o_ref[...] = h2 * dis_ref[...]

    return pl.pallas_call(
        body,
        out_shape=jax.ShapeDtypeStruct((N_NODES, D), jnp.float32),
    )(s1, h1p, dis, b1, W2)


def _tc_final(s2, h2p, dis, b2):
    def body(s_ref, h2p_ref, dis_ref, b2_ref, o_ref):
        s = (s_ref[0:N_NODES, :] + s_ref[N_PAD:N_PAD + N_NODES, :]
             + h2p_ref[...])
        o_ref[...] = dis_ref[...] * s + b2_ref[...][None, :]

    return pl.pallas_call(
        body,
        out_shape=jax.ShapeDtypeStruct((N_NODES, D), jnp.float32),
    )(s2, h2p, dis, b2)


def kernel(x, edge_index, W1, b1, W2, b2):
    src = edge_index[0].astype(jnp.int32)
    dst = edge_index[1].astype(jnp.int32)

    def pad_edges(v, e_pad, base):
        npad = e_pad - E
        pad_ar = jnp.arange(npad, dtype=jnp.int32)
        fill = pad_ar % base[0] + base[1]
        return jnp.concatenate([v, fill])

    dst_t = pad_edges(dst, E_PAD_D,
                      (N_PAD - N_NODES, N_NODES)).reshape(NW, NCH_D, CH_D)
    src_s = pad_edges(src, E_PAD_S, (N_NODES, 0)).reshape(NW, NCH_S, CH_S)
    dst_s = pad_edges(dst, E_PAD_S,
                      (N_PAD - N_NODES, N_NODES)).reshape(NW, NCH_S, CH_S)
    idx_t = jnp.stack([src_s, dst_s], axis=2)

    zeros_big = jnp.zeros((N_PAD, D), jnp.float32)
    zeros_loc = jnp.zeros((NCH_D, CH_D), jnp.float32)

    deg_p = _sc_degree(dst_t, zeros_loc)
    h1p, dis = _tc_matmul_scale(x, W1, deg_p.reshape(NW, NCH_D * CH_D))
    s1 = _sc_segsum(h1p, idx_t, zeros_big)
    h2p = _tc_mid(s1, h1p, dis, b1, W2)
    s2 = _sc_segsum(h2p, idx_t, zeros_big)
    return _tc_final(s2, h2p, dis, b2)
